# baseline (speedup 1.0000x reference)
import functools

import jax
import jax.numpy as jnp
from jax.experimental import pallas as pl
from jax.experimental.pallas import tpu as pltpu

_TBLK = 1024
_K = 2048
_W = 64


def _vq_kernel(x_ref, cb2_ref, cb_ref, cbt_ref, kf_ref, tok_ref, deq_ref,
               s1_ref, s2_ref, fit_ref):
    x = x_ref[0]
    cb2 = cb2_ref[...]
    cb = cb_ref[...]
    cbt = cbt_ref[...]
    kf = kf_ref[...]

    scores2 = jax.lax.dot_general(
        cb2, x, (((1,), (0,)), ((), ())),
        preferred_element_type=jnp.float32,
        precision=jax.lax.Precision.DEFAULT)
    xn = jnp.sum(x * x, axis=0, keepdims=True)
    cn = jnp.sum(cb * cb, axis=1, keepdims=True)
    dist = (xn - scores2) + cn

    mind = jnp.min(dist, axis=0, keepdims=True)
    mask = dist <= mind
    tokf = jnp.min(jnp.where(mask, kf, float(_K)), axis=0, keepdims=True)
    tok = tokf.astype(jnp.int32)

    onehot = jnp.where(kf == tokf, 1.0, 0.0)
    deq = jax.lax.dot_general(
        cbt, onehot, (((1,), (0,)), ((), ())),
        preferred_element_type=jnp.float32,
        precision=jax.lax.Precision.DEFAULT)

    tok_ref[0, 0] = tok
    deq_ref[0] = deq

    @pl.when((pl.program_id(0) == 0) & (pl.program_id(1) == 0))
    def _init():
        s1_ref[...] = jnp.zeros_like(s1_ref)
        s2_ref[...] = jnp.zeros_like(s2_ref)
        fit_ref[...] = jnp.zeros_like(fit_ref)

    s1_ref[...] += jnp.sum(x).reshape(1, 1)
    s2_ref[...] += jnp.sum(xn).reshape(1, 1)
    fit_ref[...] += jnp.sum(mind).reshape(1, 1)


@functools.partial(jax.jit, static_argnames=())
def kernel(hidden_states, codebook):
    B, W, S = hidden_states.shape
    K = codebook.shape[0]
    nt = S // _TBLK
    grid = (B, nt)

    scal = jax.ShapeDtypeStruct((1, 1), jnp.float32)
    scal_spec = pl.BlockSpec((1, 1), lambda b, t: (0, 0))
    kf = jnp.arange(K, dtype=jnp.float32).reshape(K, 1)

    tok4, deq, s1, s2, fit_s = pl.pallas_call(
        _vq_kernel,
        grid=grid,
        in_specs=[
            pl.BlockSpec((1, W, _TBLK), lambda b, t: (b, 0, t)),
            pl.BlockSpec((K, W), lambda b, t: (0, 0)),
            pl.BlockSpec((K, W), lambda b, t: (0, 0)),
            pl.BlockSpec((W, K), lambda b, t: (0, 0)),
            pl.BlockSpec((K, 1), lambda b, t: (0, 0)),
        ],
        out_specs=[
            pl.BlockSpec((1, 1, 1, _TBLK), lambda b, t: (b, t, 0, 0)),
            pl.BlockSpec((1, W, _TBLK), lambda b, t: (b, 0, t)),
            scal_spec, scal_spec, scal_spec,
        ],
        out_shape=[
            jax.ShapeDtypeStruct((B, nt, 1, _TBLK), jnp.int32),
            jax.ShapeDtypeStruct((B, W, S), jnp.float32),
            scal, scal, scal,
        ],
    )(hidden_states, codebook * 2.0, codebook, codebook.T, kf)

    n_total = float(B * W * S)
    n_rows = float(B * S)
    s1 = s1[0, 0]
    s2 = s2[0, 0]
    prenorm = jnp.sqrt(jnp.maximum(s2 - s1 * s1 / n_total, 0.0) / n_total)
    fit = fit_s[0, 0] / n_rows
    commit_loss = fit_s[0, 0] / n_total
    music_tokens = tok4.reshape(B, S)
    return (music_tokens, deq, commit_loss, fit, prenorm)


# speedup vs baseline: 1.0451x; 1.0451x over previous
import functools

import jax
import jax.numpy as jnp
from jax.experimental import pallas as pl

_TBLK = 2048
_K = 2048
_W = 64


def _vq_kernel(x_ref, cb2_ref, cb_ref, cbt_ref, kf_ref, tok_ref, deq_ref,
               s1_ref, s2_ref, fit_ref):
    x = x_ref[0]
    cb2 = cb2_ref[...]
    cb = cb_ref[...]
    cbt = cbt_ref[...]
    kf = kf_ref[...]

    scores2 = jax.lax.dot_general(
        cb2, x, (((1,), (0,)), ((), ())),
        preferred_element_type=jnp.float32,
        precision=jax.lax.Precision.DEFAULT)
    xn = jnp.sum(x * x, axis=0, keepdims=True)
    cn = jnp.sum(cb * cb, axis=1, keepdims=True)
    dist = (xn - scores2) + cn

    mind = jnp.min(dist, axis=0, keepdims=True)
    mask = dist <= mind
    tokf = jnp.min(jnp.where(mask, kf, float(_K)), axis=0, keepdims=True)
    tok = tokf.astype(jnp.int32)

    onehot = jnp.where(kf == tokf, 1.0, 0.0)
    deq = jax.lax.dot_general(
        cbt, onehot, (((1,), (0,)), ((), ())),
        preferred_element_type=jnp.float32,
        precision=jax.lax.Precision.DEFAULT)

    tok_ref[0, 0] = tok
    deq_ref[0] = deq

    @pl.when((pl.program_id(0) == 0) & (pl.program_id(1) == 0))
    def _init():
        s1_ref[...] = jnp.zeros_like(s1_ref)
        s2_ref[...] = jnp.zeros_like(s2_ref)
        fit_ref[...] = jnp.zeros_like(fit_ref)

    s1_ref[...] += jnp.sum(x).reshape(1, 1)
    s2_ref[...] += jnp.sum(xn).reshape(1, 1)
    fit_ref[...] += jnp.sum(mind).reshape(1, 1)


@functools.partial(jax.jit, static_argnames=())
def kernel(hidden_states, codebook):
    B, W, S = hidden_states.shape
    K = codebook.shape[0]
    nt = S // _TBLK
    grid = (B, nt)

    scal = jax.ShapeDtypeStruct((1, 1), jnp.float32)
    scal_spec = pl.BlockSpec((1, 1), lambda b, t: (0, 0))
    kf = jnp.arange(K, dtype=jnp.float32).reshape(K, 1)

    tok4, deq, s1, s2, fit_s = pl.pallas_call(
        _vq_kernel,
        grid=grid,
        in_specs=[
            pl.BlockSpec((1, W, _TBLK), lambda b, t: (b, 0, t)),
            pl.BlockSpec((K, W), lambda b, t: (0, 0)),
            pl.BlockSpec((K, W), lambda b, t: (0, 0)),
            pl.BlockSpec((W, K), lambda b, t: (0, 0)),
            pl.BlockSpec((K, 1), lambda b, t: (0, 0)),
        ],
        out_specs=[
            pl.BlockSpec((1, 1, 1, _TBLK), lambda b, t: (b, t, 0, 0)),
            pl.BlockSpec((1, W, _TBLK), lambda b, t: (b, 0, t)),
            scal_spec, scal_spec, scal_spec,
        ],
        out_shape=[
            jax.ShapeDtypeStruct((B, nt, 1, _TBLK), jnp.int32),
            jax.ShapeDtypeStruct((B, W, S), jnp.float32),
            scal, scal, scal,
        ],
    )(hidden_states, codebook * 2.0, codebook, codebook.T, kf)

    n_total = float(B * W * S)
    n_rows = float(B * S)
    s1 = s1[0, 0]
    s2 = s2[0, 0]
    prenorm = jnp.sqrt(jnp.maximum(s2 - s1 * s1 / n_total, 0.0) / n_total)
    fit = fit_s[0, 0] / n_rows
    commit_loss = fit_s[0, 0] / n_total
    music_tokens = tok4.reshape(B, S)
    return (music_tokens, deq, commit_loss, fit, prenorm)
